# baseline (speedup 1.0000x reference)
import functools

import jax
import jax.numpy as jnp
from jax import lax
from jax.experimental import pallas as pl
from jax.experimental.pallas import tpu as pltpu
from jax.experimental.pallas import tpu_sc as plsc

BATCH = 4096
MAX_LEN = 200
WORD_DIM = 64
N_LABELS = 100
PK = WORD_DIM // 2
RB = MAX_LEN * PK // 128
OUT_ROWS = BATCH * RB
NW = 32
B_PER_W = BATCH // NW
G1 = 104
G2 = MAX_LEN - G1


def _sc_gather(table, idx):
    mesh = plsc.VectorSubcoreMesh(core_axis_name="c", subcore_axis_name="s")

    @functools.partial(
        pl.kernel,
        out_type=jax.ShapeDtypeStruct((OUT_ROWS, 128), jnp.float32),
        mesh=mesh,
        compiler_params=pltpu.CompilerParams(use_tc_tiling_on_sc=False),
        scratch_types=[
            pltpu.VMEM((B_PER_W, MAX_LEN), jnp.int32),
            pltpu.VMEM((2, MAX_LEN, PK), jnp.float32),
            pltpu.VMEM((2, RB, 128), jnp.float32),
            pltpu.SemaphoreType.DMA,
            pltpu.SemaphoreType.DMA,
        ],
    )
    def k(table_hbm, idx_hbm, out_hbm, idx_v, ga_v, gb_v, gsem, osem):
        wid = lax.axis_index("s") * 2 + lax.axis_index("c")
        b0 = wid * B_PER_W
        pltpu.sync_copy(idx_hbm.at[pl.ds(b0, B_PER_W)], idx_v)

        def gather_pair(i, slot):
            return (
                pltpu.make_async_copy(
                    table_hbm.at[idx_v.at[i, pl.ds(0, G1)]],
                    ga_v.at[slot, pl.ds(0, G1)], gsem),
                pltpu.make_async_copy(
                    table_hbm.at[idx_v.at[i, pl.ds(G1, G2)]],
                    ga_v.at[slot, pl.ds(G1, G2)], gsem),
            )

        def out_copy(i, slot):
            return pltpu.make_async_copy(
                gb_v.at[slot],
                out_hbm.at[pl.ds((b0 + i) * RB, RB)], osem)

        for c in gather_pair(0, 0):
            c.start()

        def body(i, _):
            slot = lax.rem(i, 2)
            for c in gather_pair(i, slot):
                c.wait()

            @pl.when(i + 1 < B_PER_W)
            def _():
                for c in gather_pair(i + 1, 1 - slot):
                    c.start()

            @pl.when(i >= 2)
            def _():
                out_copy(i - 2, slot).wait()

            for w in range(0, MAX_LEN * PK, 16):
                gb_v[slot, w // 128, pl.ds(w % 128, 16)] = (
                    ga_v[slot, w // PK, pl.ds(w % PK, 16)])

            out_copy(i, slot).start()
            return 0

        lax.fori_loop(0, B_PER_W, body, 0)
        out_copy(B_PER_W - 2, 0).wait()
        out_copy(B_PER_W - 1, 1).wait()

    return k(table, idx)


def _tc_matmul(x, Wr, b2d):
    BB = 256

    def mm(x_ref, w_ref, b_ref, o_ref):
        xb = pltpu.bitcast(x_ref[...], jnp.bfloat16)
        x3 = xb.reshape(BB, 2 * RB, 128)
        acc = jnp.zeros((BB, N_LABELS), jnp.float32)
        for j in range(2 * RB):
            acc += jnp.dot(x3[:, j, :], w_ref[j],
                           preferred_element_type=jnp.float32)
        o_ref[...] = acc + b_ref[...]

    return pl.pallas_call(
        mm,
        grid=(BATCH // BB,),
        in_specs=[
            pl.BlockSpec((BB * RB, 128), lambda i: (i, 0)),
            pl.BlockSpec((2 * RB, 128, N_LABELS), lambda i: (0, 0, 0)),
            pl.BlockSpec((1, N_LABELS), lambda i: (0, 0)),
        ],
        out_specs=pl.BlockSpec((BB, N_LABELS), lambda i: (i, 0)),
        out_shape=jax.ShapeDtypeStruct((BATCH, N_LABELS), jnp.float32),
    )(x, Wr, b2d)


def kernel(input, table, W, b):
    tpack = lax.bitcast_convert_type(
        table.astype(jnp.bfloat16).reshape(-1, PK, 2), jnp.float32)
    flat = _sc_gather(tpack, input.astype(jnp.int32))
    Wr = (W.reshape(N_LABELS, RB, 128, 2).transpose(1, 3, 2, 0)
          .reshape(2 * RB, 128, N_LABELS).astype(jnp.bfloat16))
    return _tc_matmul(flat, Wr, b.reshape(1, N_LABELS))


# speedup vs baseline: 1.3582x; 1.3582x over previous
import functools

import jax
import jax.numpy as jnp
from jax import lax
from jax.experimental import pallas as pl
from jax.experimental.pallas import tpu as pltpu
from jax.experimental.pallas import tpu_sc as plsc

BATCH = 4096
MAX_LEN = 200
WORD_DIM = 64
N_LABELS = 100
PK = WORD_DIM // 2
RB = MAX_LEN * PK // 128
OUT_ROWS = BATCH * RB
NW = 32
B_PER_W = BATCH // NW
G1 = 104
G2 = MAX_LEN - G1


def _sc_gather(table, idx):
    mesh = plsc.VectorSubcoreMesh(core_axis_name="c", subcore_axis_name="s")

    @functools.partial(
        pl.kernel,
        out_type=jax.ShapeDtypeStruct((OUT_ROWS, 128), jnp.float32),
        mesh=mesh,
        compiler_params=pltpu.CompilerParams(use_tc_tiling_on_sc=False),
        scratch_types=[
            pltpu.VMEM((B_PER_W, MAX_LEN), jnp.int32),
            pltpu.VMEM((2, MAX_LEN, PK), jnp.float32),
            pltpu.VMEM((2, RB, 128), jnp.float32),
            pltpu.SemaphoreType.DMA,
            pltpu.SemaphoreType.DMA,
        ],
    )
    def k(table_hbm, idx_hbm, out_hbm, idx_v, ga_v, gb_v, gsem, osem):
        wid = lax.axis_index("s") * 2 + lax.axis_index("c")
        b0 = wid * B_PER_W
        pltpu.sync_copy(idx_hbm.at[pl.ds(b0, B_PER_W)], idx_v)

        def gather_pair(i, slot):
            return (
                pltpu.make_async_copy(
                    table_hbm.at[idx_v.at[i, pl.ds(0, G1)]],
                    ga_v.at[slot, pl.ds(0, G1)], gsem),
                pltpu.make_async_copy(
                    table_hbm.at[idx_v.at[i, pl.ds(G1, G2)]],
                    ga_v.at[slot, pl.ds(G1, G2)], gsem),
            )

        def out_copy(i, slot):
            return pltpu.make_async_copy(
                gb_v.at[slot],
                out_hbm.at[pl.ds((b0 + i) * RB, RB)], osem)

        for c in gather_pair(0, 0):
            c.start()

        def body(i, _):
            slot = lax.rem(i, 2)
            for c in gather_pair(i, slot):
                c.wait()

            @pl.when(i + 1 < B_PER_W)
            def _():
                for c in gather_pair(i + 1, 1 - slot):
                    c.start()

            @pl.when(i >= 2)
            def _():
                out_copy(i - 2, slot).wait()

            for w in range(0, MAX_LEN * PK, 16):
                gb_v[slot, w // 128, pl.ds(w % 128, 16)] = (
                    ga_v[slot, w // PK, pl.ds(w % PK, 16)])

            out_copy(i, slot).start()
            return 0

        lax.fori_loop(0, B_PER_W, body, 0)
        out_copy(B_PER_W - 2, 0).wait()
        out_copy(B_PER_W - 1, 1).wait()

    return k(table, idx)


def _tc_matmul(x, Wr, b2d):
    BB = 256

    def mm(x_ref, w_ref, b_ref, o_ref):
        xb = pltpu.bitcast(x_ref[...], jnp.bfloat16)
        x3 = xb.reshape(BB, 2 * RB, 128)
        acc = jnp.zeros((BB, N_LABELS), jnp.float32)
        for j in range(2 * RB):
            acc += jnp.dot(x3[:, j, :], w_ref[j],
                           preferred_element_type=jnp.float32)
        o_ref[...] = acc + b_ref[...]

    return pl.pallas_call(
        mm,
        grid=(BATCH // BB,),
        in_specs=[
            pl.BlockSpec((BB * RB, 128), lambda i: (i, 0)),
            pl.BlockSpec((2 * RB, 128, N_LABELS), lambda i: (0, 0, 0)),
            pl.BlockSpec((1, N_LABELS), lambda i: (0, 0)),
        ],
        out_specs=pl.BlockSpec((BB, N_LABELS), lambda i: (i, 0)),
        out_shape=jax.ShapeDtypeStruct((BATCH, N_LABELS), jnp.float32),
    )(x, Wr, b2d)


def kernel(input, table, W, b):
    def rnd(v):
        return lax.bitcast_convert_type(
            v.astype(jnp.bfloat16).astype(jnp.float32), jnp.uint32)
    word = (rnd(table[:, :PK]) >> 16) | (rnd(table[:, PK:]) & jnp.uint32(0xFFFF0000))
    tpack = lax.bitcast_convert_type(word, jnp.float32)
    flat = _sc_gather(tpack, input.astype(jnp.int32))
    q = jnp.arange(RB * 128).reshape(RB, 1, 128)
    p = jnp.arange(2).reshape(1, 2, 1)
    feat = WORD_DIM * (q // PK) + q % PK + PK * p
    Wr = (jnp.take(W, feat.reshape(-1), axis=1)
          .reshape(N_LABELS, 2 * RB, 128).transpose(1, 2, 0)
          .astype(jnp.bfloat16))
    return _tc_matmul(flat, Wr, b.reshape(1, N_LABELS))


# speedup vs baseline: 1.3606x; 1.0018x over previous
import functools

import jax
import jax.numpy as jnp
from jax import lax
from jax.experimental import pallas as pl
from jax.experimental.pallas import tpu as pltpu
from jax.experimental.pallas import tpu_sc as plsc

BATCH = 4096
MAX_LEN = 200
WORD_DIM = 64
N_LABELS = 100
PK = WORD_DIM // 2
RB = MAX_LEN * PK // 128
OUT_ROWS = BATCH * RB
NW = 32
B_PER_W = BATCH // NW
G1 = 104
G2 = MAX_LEN - G1


def _sc_gather(table, idx):
    mesh = plsc.VectorSubcoreMesh(core_axis_name="c", subcore_axis_name="s")

    @functools.partial(
        pl.kernel,
        out_type=jax.ShapeDtypeStruct((OUT_ROWS, 128), jnp.float32),
        mesh=mesh,
        compiler_params=pltpu.CompilerParams(use_tc_tiling_on_sc=False),
        scratch_types=[
            pltpu.VMEM((B_PER_W, MAX_LEN), jnp.int32),
            pltpu.VMEM((2, MAX_LEN, PK), jnp.float32),
            pltpu.VMEM((2, RB, 128), jnp.float32),
            pltpu.SemaphoreType.DMA,
            pltpu.SemaphoreType.DMA,
        ],
    )
    def k(table_hbm, idx_hbm, out_hbm, idx_v, ga_v, gb_v, gsem, osem):
        wid = lax.axis_index("s") * 2 + lax.axis_index("c")
        b0 = wid * B_PER_W
        pltpu.sync_copy(idx_hbm.at[pl.ds(b0, B_PER_W)], idx_v)

        def gather_pair(i, slot):
            return (
                pltpu.make_async_copy(
                    table_hbm.at[idx_v.at[i, pl.ds(0, G1)]],
                    ga_v.at[slot, pl.ds(0, G1)], gsem),
                pltpu.make_async_copy(
                    table_hbm.at[idx_v.at[i, pl.ds(G1, G2)]],
                    ga_v.at[slot, pl.ds(G1, G2)], gsem),
            )

        def out_copy(i, slot):
            return pltpu.make_async_copy(
                gb_v.at[slot],
                out_hbm.at[pl.ds((b0 + i) * RB, RB)], osem)

        for c in gather_pair(0, 0):
            c.start()

        def body(i, _):
            slot = lax.rem(i, 2)
            for c in gather_pair(i, slot):
                c.wait()

            @pl.when(i + 1 < B_PER_W)
            def _():
                for c in gather_pair(i + 1, 1 - slot):
                    c.start()

            @pl.when(i >= 2)
            def _():
                out_copy(i - 2, slot).wait()

            for w in range(0, MAX_LEN * PK, 16):
                gb_v[slot, w // 128, pl.ds(w % 128, 16)] = (
                    ga_v[slot, w // PK, pl.ds(w % PK, 16)])

            out_copy(i, slot).start()
            return 0

        lax.fori_loop(0, B_PER_W, body, 0)
        out_copy(B_PER_W - 2, 0).wait()
        out_copy(B_PER_W - 1, 1).wait()

    return k(table, idx)


def _tc_matmul(x, Wr, b2d):
    BB = 256

    def mm(x_ref, w_ref, b_ref, o_ref):
        xb = pltpu.bitcast(x_ref[...], jnp.bfloat16)
        x3 = xb.reshape(BB, 2 * RB, 128)
        acc = jnp.zeros((BB, N_LABELS), jnp.float32)
        for j in range(2 * RB):
            acc += jnp.dot(x3[:, j, :], w_ref[j],
                           preferred_element_type=jnp.float32)
        o_ref[...] = acc + b_ref[...]

    return pl.pallas_call(
        mm,
        grid=(BATCH // BB,),
        in_specs=[
            pl.BlockSpec((BB * RB, 128), lambda i: (i, 0)),
            pl.BlockSpec((2 * RB, 128, N_LABELS), lambda i: (0, 0, 0)),
            pl.BlockSpec((1, N_LABELS), lambda i: (0, 0)),
        ],
        out_specs=pl.BlockSpec((BB, N_LABELS), lambda i: (i, 0)),
        out_shape=jax.ShapeDtypeStruct((BATCH, N_LABELS), jnp.float32),
    )(x, Wr, b2d)


def kernel(input, table, W, b):
    def rnd(v):
        return lax.bitcast_convert_type(
            v.astype(jnp.bfloat16).astype(jnp.float32), jnp.uint32)
    word = (rnd(table[:, :PK]) >> 16) | (rnd(table[:, PK:]) & jnp.uint32(0xFFFF0000))
    tpack = lax.bitcast_convert_type(word, jnp.float32)
    flat = _sc_gather(tpack, input.astype(jnp.int32))
    Wr = (W.reshape(N_LABELS, RB, 4, 2, PK).transpose(1, 3, 2, 4, 0)
          .reshape(2 * RB, 128, N_LABELS).astype(jnp.bfloat16))
    return _tc_matmul(flat, Wr, b.reshape(1, N_LABELS))


# speedup vs baseline: 1.5932x; 1.1710x over previous
import functools

import jax
import jax.numpy as jnp
from jax import lax
from jax.experimental import pallas as pl
from jax.experimental.pallas import tpu as pltpu
from jax.experimental.pallas import tpu_sc as plsc

BATCH = 4096
MAX_LEN = 200
WORD_DIM = 64
N_LABELS = 100
PK = WORD_DIM // 2
RB = MAX_LEN * PK // 128
NW = 32
NC = 2
CB = BATCH // NC
B_PER_W = CB // NW
G1 = 104
G2 = MAX_LEN - G1


def _sc_gather(table, idx):
    mesh = plsc.VectorSubcoreMesh(core_axis_name="c", subcore_axis_name="s")

    @functools.partial(
        pl.kernel,
        out_type=jax.ShapeDtypeStruct((CB * RB, 128), jnp.float32),
        mesh=mesh,
        compiler_params=pltpu.CompilerParams(use_tc_tiling_on_sc=False),
        scratch_types=[
            pltpu.VMEM((B_PER_W, MAX_LEN), jnp.int32),
            pltpu.VMEM((2, MAX_LEN, PK), jnp.float32),
            pltpu.VMEM((2, RB, 128), jnp.float32),
            pltpu.SemaphoreType.DMA,
            pltpu.SemaphoreType.DMA,
        ],
    )
    def k(table_hbm, idx_hbm, out_hbm, idx_v, ga_v, gb_v, gsem, osem):
        wid = lax.axis_index("s") * 2 + lax.axis_index("c")
        b0 = wid * B_PER_W
        pltpu.sync_copy(idx_hbm.at[pl.ds(b0, B_PER_W)], idx_v)

        def gather_pair(i, slot):
            return (
                pltpu.make_async_copy(
                    table_hbm.at[idx_v.at[i, pl.ds(0, G1)]],
                    ga_v.at[slot, pl.ds(0, G1)], gsem),
                pltpu.make_async_copy(
                    table_hbm.at[idx_v.at[i, pl.ds(G1, G2)]],
                    ga_v.at[slot, pl.ds(G1, G2)], gsem),
            )

        def out_copy(i, slot):
            return pltpu.make_async_copy(
                gb_v.at[slot],
                out_hbm.at[pl.ds((b0 + i) * RB, RB)], osem)

        for c in gather_pair(0, 0):
            c.start()

        def body(i, _):
            slot = lax.rem(i, 2)
            for c in gather_pair(i, slot):
                c.wait()

            @pl.when(i + 1 < B_PER_W)
            def _():
                for c in gather_pair(i + 1, 1 - slot):
                    c.start()

            @pl.when(i >= 2)
            def _():
                out_copy(i - 2, slot).wait()

            for w in range(0, MAX_LEN * PK, 16):
                gb_v[slot, w // 128, pl.ds(w % 128, 16)] = (
                    ga_v[slot, w // PK, pl.ds(w % PK, 16)])

            out_copy(i, slot).start()
            return 0

        lax.fori_loop(0, B_PER_W, body, 0)
        out_copy(B_PER_W - 2, 0).wait()
        out_copy(B_PER_W - 1, 1).wait()

    return k(table, idx)


def _tc_matmul(x, Wr, b2d):
    BB = 256

    def mm(x_ref, w_ref, b_ref, o_ref):
        xb = pltpu.bitcast(x_ref[...], jnp.bfloat16)
        x3 = xb.reshape(BB, 2 * RB, 128)
        acc = jnp.zeros((BB, N_LABELS), jnp.float32)
        for j in range(2 * RB):
            acc += jnp.dot(x3[:, j, :], w_ref[j],
                           preferred_element_type=jnp.float32)
        o_ref[...] = acc + b_ref[...]

    return pl.pallas_call(
        mm,
        grid=(CB // BB,),
        in_specs=[
            pl.BlockSpec((BB * RB, 128), lambda i: (i, 0)),
            pl.BlockSpec((2 * RB, 128, N_LABELS), lambda i: (0, 0, 0)),
            pl.BlockSpec((1, N_LABELS), lambda i: (0, 0)),
        ],
        out_specs=pl.BlockSpec((BB, N_LABELS), lambda i: (i, 0)),
        out_shape=jax.ShapeDtypeStruct((CB, N_LABELS), jnp.float32),
    )(x, Wr, b2d)


def kernel(input, table, W, b):
    def rnd(v):
        return lax.bitcast_convert_type(
            v.astype(jnp.bfloat16).astype(jnp.float32), jnp.uint32)
    word = (rnd(table[:, :PK]) >> 16) | (rnd(table[:, PK:]) & jnp.uint32(0xFFFF0000))
    tpack = lax.bitcast_convert_type(word, jnp.float32)
    Wr = (W.reshape(N_LABELS, RB, 4, 2, PK).transpose(1, 3, 2, 4, 0)
          .reshape(2 * RB, 128, N_LABELS).astype(jnp.bfloat16))
    b2d = b.reshape(1, N_LABELS)
    idx = input.astype(jnp.int32)
    flats = [_sc_gather(tpack, idx[c * CB:(c + 1) * CB]) for c in range(NC)]
    outs = [_tc_matmul(f, Wr, b2d) for f in flats]
    return jnp.concatenate(outs, axis=0)


# speedup vs baseline: 1.6909x; 1.0613x over previous
import functools

import jax
import jax.numpy as jnp
from jax import lax
from jax.experimental import pallas as pl
from jax.experimental.pallas import tpu as pltpu
from jax.experimental.pallas import tpu_sc as plsc

BATCH = 4096
MAX_LEN = 200
WORD_DIM = 64
N_LABELS = 100
PK = WORD_DIM // 2
RB = MAX_LEN * PK // 128
NW = 32
NC = 4
CB = BATCH // NC
B_PER_W = CB // NW
G1 = 104
G2 = MAX_LEN - G1


def _sc_gather(table, idx):
    mesh = plsc.VectorSubcoreMesh(core_axis_name="c", subcore_axis_name="s")

    @functools.partial(
        pl.kernel,
        out_type=jax.ShapeDtypeStruct((CB * RB, 128), jnp.float32),
        mesh=mesh,
        compiler_params=pltpu.CompilerParams(use_tc_tiling_on_sc=False),
        scratch_types=[
            pltpu.VMEM((B_PER_W, MAX_LEN), jnp.int32),
            pltpu.VMEM((2, MAX_LEN, PK), jnp.float32),
            pltpu.VMEM((2, RB, 128), jnp.float32),
            pltpu.SemaphoreType.DMA,
            pltpu.SemaphoreType.DMA,
        ],
    )
    def k(table_hbm, idx_hbm, out_hbm, idx_v, ga_v, gb_v, gsem, osem):
        wid = lax.axis_index("s") * 2 + lax.axis_index("c")
        b0 = wid * B_PER_W
        pltpu.sync_copy(idx_hbm.at[pl.ds(b0, B_PER_W)], idx_v)

        def gather_pair(i, slot):
            return (
                pltpu.make_async_copy(
                    table_hbm.at[idx_v.at[i, pl.ds(0, G1)]],
                    ga_v.at[slot, pl.ds(0, G1)], gsem),
                pltpu.make_async_copy(
                    table_hbm.at[idx_v.at[i, pl.ds(G1, G2)]],
                    ga_v.at[slot, pl.ds(G1, G2)], gsem),
            )

        def out_copy(i, slot):
            return pltpu.make_async_copy(
                gb_v.at[slot],
                out_hbm.at[pl.ds((b0 + i) * RB, RB)], osem)

        for c in gather_pair(0, 0):
            c.start()

        def body(i, _):
            slot = lax.rem(i, 2)
            for c in gather_pair(i, slot):
                c.wait()

            @pl.when(i + 1 < B_PER_W)
            def _():
                for c in gather_pair(i + 1, 1 - slot):
                    c.start()

            @pl.when(i >= 2)
            def _():
                out_copy(i - 2, slot).wait()

            for w in range(0, MAX_LEN * PK, 16):
                gb_v[slot, w // 128, pl.ds(w % 128, 16)] = (
                    ga_v[slot, w // PK, pl.ds(w % PK, 16)])

            out_copy(i, slot).start()
            return 0

        lax.fori_loop(0, B_PER_W, body, 0)
        out_copy(B_PER_W - 2, 0).wait()
        out_copy(B_PER_W - 1, 1).wait()

    return k(table, idx)


def _tc_matmul(x, Wr, b2d):
    BB = 256

    def mm(x_ref, w_ref, b_ref, o_ref):
        xb = pltpu.bitcast(x_ref[...], jnp.bfloat16)
        x3 = xb.reshape(BB, 2 * RB, 128)
        acc = jnp.zeros((BB, N_LABELS), jnp.float32)
        for j in range(2 * RB):
            acc += jnp.dot(x3[:, j, :], w_ref[j],
                           preferred_element_type=jnp.float32)
        o_ref[...] = acc + b_ref[...]

    return pl.pallas_call(
        mm,
        grid=(CB // BB,),
        in_specs=[
            pl.BlockSpec((BB * RB, 128), lambda i: (i, 0)),
            pl.BlockSpec((2 * RB, 128, N_LABELS), lambda i: (0, 0, 0)),
            pl.BlockSpec((1, N_LABELS), lambda i: (0, 0)),
        ],
        out_specs=pl.BlockSpec((BB, N_LABELS), lambda i: (i, 0)),
        out_shape=jax.ShapeDtypeStruct((CB, N_LABELS), jnp.float32),
    )(x, Wr, b2d)


def kernel(input, table, W, b):
    def rnd(v):
        return lax.bitcast_convert_type(
            v.astype(jnp.bfloat16).astype(jnp.float32), jnp.uint32)
    word = (rnd(table[:, :PK]) >> 16) | (rnd(table[:, PK:]) & jnp.uint32(0xFFFF0000))
    tpack = lax.bitcast_convert_type(word, jnp.float32)
    Wr = (W.reshape(N_LABELS, RB, 4, 2, PK).transpose(1, 3, 2, 4, 0)
          .reshape(2 * RB, 128, N_LABELS).astype(jnp.bfloat16))
    b2d = b.reshape(1, N_LABELS)
    idx = input.astype(jnp.int32)
    flats = [_sc_gather(tpack, idx[c * CB:(c + 1) * CB]) for c in range(NC)]
    outs = [_tc_matmul(f, Wr, b2d) for f in flats]
    return jnp.concatenate(outs, axis=0)
